# baseline (speedup 1.0000x reference)
import functools

import jax
import jax.numpy as jnp
from jax import lax
from jax.experimental import pallas as pl
from jax.experimental.pallas import tpu as pltpu
from jax.experimental.pallas import tpu_sc as plsc


def _build_emb_kernel(N, S, D, n_cores, n_subcores, chunk, nbuf):
    n_workers = n_cores * n_subcores
    B = N // S
    s_per_w = S // n_workers
    n_sc = s_per_w // chunk
    n_steps = n_sc * B
    n_per_w = B * s_per_w
    mesh = plsc.VectorSubcoreMesh(core_axis_name="c", subcore_axis_name="s")

    scratch = [pltpu.VMEM((n_per_w,), jnp.int32)]
    scratch += [pltpu.VMEM((chunk, D), jnp.float32) for _ in range(nbuf + 2)]
    scratch += [pltpu.SemaphoreType.DMA for _ in range(2 * nbuf + 3)]

    @functools.partial(
        pl.kernel,
        mesh=mesh,
        out_type=jax.ShapeDtypeStruct((N, D), jnp.float32),
        scratch_types=scratch,
    )
    def emb_kernel(ids_hbm, wemb_hbm, pemb_hbm, out_hbm, idx_v, *bufs):
        rows = bufs[0:nbuf]
        pos = bufs[nbuf:nbuf + 2]
        sem_g = bufs[nbuf + 2:2 * nbuf + 2]
        sem_o = bufs[2 * nbuf + 2:3 * nbuf + 2]
        sem_p = bufs[3 * nbuf + 2:3 * nbuf + 4]
        sem_i = bufs[3 * nbuf + 4]
        wid = lax.axis_index("s") * n_cores + lax.axis_index("c")
        base_s = wid * s_per_w

        idx_copies = [
            pltpu.async_copy(
                ids_hbm.at[pl.ds(b * S + base_s, s_per_w)],
                idx_v.at[pl.ds(b * s_per_w, s_per_w)],
                sem_i,
            )
            for b in range(B)
        ]
        for cp in idx_copies:
            cp.wait()

        def start_gather(j):
            c, b = j // B, j % B
            rb = j % nbuf
            return pltpu.async_copy(
                wemb_hbm.at[idx_v.at[pl.ds(b * s_per_w + c * chunk, chunk)]],
                rows[rb],
                sem_g[rb],
            )

        def start_pos(c):
            return pltpu.async_copy(
                pemb_hbm.at[pl.ds(base_s + c * chunk, chunk)], pos[c % 2],
                sem_p[c % 2],
            )

        gather_flight = {}
        pos_flight = {}
        out_flight = {}
        for j in range(min(nbuf - 1, n_steps)):
            gather_flight[j] = start_gather(j)
        for c in range(min(2, n_sc)):
            pos_flight[c] = start_pos(c)

        for j in range(n_steps):
            c, b = j // B, j % B
            rb = j % nbuf
            pc = c % 2
            gather_flight.pop(j).wait()
            if b == 0:
                pos_flight.pop(c).wait()
                if c >= 1 and c + 1 < n_sc:
                    pos_flight[c + 1] = start_pos(c + 1)
            nxt = j + nbuf - 1
            if nxt < n_steps:
                if j >= 1:
                    out_flight.pop(j - 1).wait()
                gather_flight[nxt] = start_gather(nxt)

            @plsc.parallel_loop(0, chunk)
            def add_row(r):
                for k in range(D // 16):
                    sl = pl.ds(k * 16, 16)
                    plsc.addupdate(rows[rb].at[r, sl], pos[pc][r, sl])
            out_flight[j] = pltpu.async_copy(
                rows[rb],
                out_hbm.at[pl.ds(b * S + base_s + c * chunk, chunk)],
                sem_o[rb],
            )
        for j, o in out_flight.items():
            o.wait()

    return emb_kernel


def kernel(input_ids, word_embeddings, position_embeddings):
    B, S = input_ids.shape
    V, D = word_embeddings.shape
    N = B * S
    info = plsc.get_sparse_core_info()
    ids = input_ids.reshape(N).astype(jnp.int32)
    emb = _build_emb_kernel(
        N, S, D, info.num_cores, info.num_subcores, chunk=16, nbuf=4
    )
    out = emb(ids, word_embeddings, position_embeddings)
    return out.reshape(B, S, D)


# speedup vs baseline: 1.0562x; 1.0562x over previous
import functools

import jax
import jax.numpy as jnp
from jax import lax
from jax.experimental import pallas as pl
from jax.experimental.pallas import tpu as pltpu
from jax.experimental.pallas import tpu_sc as plsc


def _build_emb_kernel(N, S, D, n_cores, n_subcores, chunk, nbuf):
    n_workers = n_cores * n_subcores
    B = N // S
    s_per_w = S // n_workers
    n_sc = s_per_w // chunk
    n_steps = n_sc * B
    n_per_w = B * s_per_w
    mesh = plsc.VectorSubcoreMesh(core_axis_name="c", subcore_axis_name="s")

    scratch = [pltpu.VMEM((n_per_w,), jnp.int32)]
    scratch += [pltpu.VMEM((chunk, D), jnp.float32) for _ in range(nbuf + 2)]
    scratch += [pltpu.SemaphoreType.DMA for _ in range(2 * nbuf + 3)]

    @functools.partial(
        pl.kernel,
        mesh=mesh,
        out_type=jax.ShapeDtypeStruct((N, D), jnp.float32),
        scratch_types=scratch,
    )
    def emb_kernel(ids_hbm, wemb_hbm, pemb_hbm, out_hbm, idx_v, *bufs):
        rows = bufs[0:nbuf]
        pos = bufs[nbuf:nbuf + 2]
        sem_g = bufs[nbuf + 2:2 * nbuf + 2]
        sem_o = bufs[2 * nbuf + 2:3 * nbuf + 2]
        sem_p = bufs[3 * nbuf + 2:3 * nbuf + 4]
        sem_i = bufs[3 * nbuf + 4]
        wid = lax.axis_index("s") * n_cores + lax.axis_index("c")
        base_s = wid * s_per_w

        idx_copies = [
            pltpu.async_copy(
                ids_hbm.at[pl.ds(b * S + base_s, s_per_w)],
                idx_v.at[pl.ds(b * s_per_w, s_per_w)],
                sem_i,
            )
            for b in range(B)
        ]
        for cp in idx_copies:
            cp.wait()

        def start_gather(j):
            c, b = j // B, j % B
            rb = j % nbuf
            return pltpu.async_copy(
                wemb_hbm.at[idx_v.at[pl.ds(b * s_per_w + c * chunk, chunk)]],
                rows[rb],
                sem_g[rb],
            )

        def start_pos(c):
            return pltpu.async_copy(
                pemb_hbm.at[pl.ds(base_s + c * chunk, chunk)], pos[c % 2],
                sem_p[c % 2],
            )

        gather_flight = {}
        pos_flight = {}
        out_flight = {}
        for j in range(min(nbuf - 1, n_steps)):
            gather_flight[j] = start_gather(j)
        for c in range(min(2, n_sc)):
            pos_flight[c] = start_pos(c)

        for j in range(n_steps):
            c, b = j // B, j % B
            rb = j % nbuf
            pc = c % 2
            gather_flight.pop(j).wait()
            if b == 0:
                pos_flight.pop(c).wait()
                if c >= 1 and c + 1 < n_sc:
                    pos_flight[c + 1] = start_pos(c + 1)
            nxt = j + nbuf - 1
            if nxt < n_steps:
                if j >= 1:
                    out_flight.pop(j - 1).wait()
                gather_flight[nxt] = start_gather(nxt)

            def add_row(r, _):
                for k in range(D // 16):
                    sl = pl.ds(k * 16, 16)
                    plsc.addupdate(rows[rb].at[r, sl], pos[pc][r, sl])
                return 0

            lax.fori_loop(0, chunk, add_row, 0)
            out_flight[j] = pltpu.async_copy(
                rows[rb],
                out_hbm.at[pl.ds(b * S + base_s + c * chunk, chunk)],
                sem_o[rb],
            )
        for j, o in out_flight.items():
            o.wait()

    return emb_kernel


def kernel(input_ids, word_embeddings, position_embeddings):
    B, S = input_ids.shape
    V, D = word_embeddings.shape
    N = B * S
    info = plsc.get_sparse_core_info()
    ids = input_ids.reshape(N).astype(jnp.int32)
    emb = _build_emb_kernel(
        N, S, D, info.num_cores, info.num_subcores, chunk=16, nbuf=4
    )
    out = emb(ids, word_embeddings, position_embeddings)
    return out.reshape(B, S, D)
